# baseline (speedup 1.0000x reference)
import functools

import jax
import jax.numpy as jnp
from jax import lax
from jax.experimental import pallas as pl
from jax.experimental.pallas import tpu as pltpu
from jax.experimental.pallas import tpu_sc as plsc

_EPS = 1e-16
_SEL0 = 64
_SEL1 = 128
_D = _SEL0 + _SEL1
_N = 1024
_K = 128
_NREG = _K // 16
_BLK = 16
_RB = 256


def _dist_kernel(xi_ref, xt_ref, tf_ref, box_ref, out_ref):
    rb = pl.program_id(1)
    box = box_ref[0, 0]
    inv_box = 1.0 / box
    xi = xi_ref[0]
    acc = jnp.zeros((_RB, _N), jnp.float32)
    for k in range(3):
        t = xi[:, k : k + 1] - xt_ref[0, k : k + 1, :] + _EPS
        t = t - box * jnp.round(t * inv_box)
        acc = acc + t * t
    row_i = jax.lax.broadcasted_iota(jnp.int32, (_RB, _N), 0) + rb * _RB
    col_j = jax.lax.broadcasted_iota(jnp.int32, (_RB, _N), 1)
    diag = row_i == col_j
    is0 = tf_ref[0] == 0.0
    inf = jnp.float32(jnp.inf)
    out_ref[0, 0] = jnp.where(diag | (~is0), inf, acc)
    out_ref[0, 1] = jnp.where(diag | is0, inf, acc)


def _s16(v):
    r = plsc.sort_key_val(v, v)
    return r[0] if isinstance(r, (tuple, list)) else r


def _bitonic_fix(regs):
    regs = list(regs)
    n = len(regs)
    d = n // 2
    while d >= 1:
        for base in range(0, n, 2 * d):
            for off in range(d):
                a = regs[base + off]
                b = regs[base + off + d]
                regs[base + off] = jnp.minimum(a, b)
                regs[base + off + d] = jnp.maximum(a, b)
        d //= 2
    return [_s16(r) for r in regs]


def _merge(a, b):
    m = len(a)
    fb = [jnp.flip(b[m - 1 - i], 0) for i in range(m)]
    lo = [jnp.minimum(a[i], fb[i]) for i in range(m)]
    hi = [jnp.maximum(a[i], fb[i]) for i in range(m)]
    return _bitonic_fix(lo), _bitonic_fix(hi)


def _sort_chunk(vs):
    rs = [_s16(v) for v in vs]
    width = 1
    while width < len(rs):
        nxt = []
        for p in range(0, len(rs), 2 * width):
            lo, hi = _merge(rs[p : p + width], rs[p + width : p + 2 * width])
            nxt.extend(lo + hi)
        rs = nxt
        width *= 2
    return rs


def _merge_keep_lo(buf, c):
    m = len(buf)
    fc = [jnp.flip(c[m - 1 - i], 0) for i in range(m)]
    lo = [jnp.minimum(buf[i], fc[i]) for i in range(m)]
    return _bitonic_fix(lo)


def _make_sc_topk(R):
    mesh = plsc.VectorSubcoreMesh(core_axis_name="c", subcore_axis_name="s")
    rpw = R // 32
    nblk = rpw // _BLK

    @functools.partial(
        pl.kernel, mesh=mesh,
        compiler_params=pltpu.CompilerParams(needs_layout_passes=False),
        out_type=jax.ShapeDtypeStruct((R * _K,), jnp.float32),
        scratch_types=[
            pltpu.VMEM((_BLK * _N,), jnp.float32),
            pltpu.VMEM((_BLK * _K,), jnp.float32),
        ],
    )
    def k(x_hbm, out_hbm, buf_v, out_v):
        wid = lax.axis_index("s") * 2 + lax.axis_index("c")
        base_row = wid * rpw

        def blk_body(bi, _):
            row0 = base_row + bi * _BLK
            pltpu.sync_copy(x_hbm.at[pl.ds(row0 * _N, _BLK * _N)], buf_v)

            def row_body(j, _):
                def vreg(t):
                    return buf_v[pl.ds(j * _N + t * 16, 16)]

                buf = _sort_chunk([vreg(t) for t in range(8)])
                for c in range(1, 8):
                    vs = [vreg(8 * c + t) for t in range(8)]
                    m = vs[0]
                    for t in range(1, 8):
                        m = jnp.minimum(m, vs[t])
                    cmin = jnp.min(m)
                    bmax = jnp.max(buf[_NREG - 1])
                    buf = tuple(lax.cond(
                        cmin < bmax,
                        lambda: tuple(_merge_keep_lo(buf, _sort_chunk(vs))),
                        lambda: tuple(buf)))
                for r in range(_NREG):
                    out_v[pl.ds(j * _K + r * 16, 16)] = buf[r]
                return 0

            lax.fori_loop(0, _BLK, row_body, 0)
            pltpu.sync_copy(out_v, out_hbm.at[pl.ds(row0 * _K, _BLK * _K)])
            return 0

        lax.fori_loop(0, nblk, blk_body, 0)

    return k


def _mlp_kernel(sq_ref, tc_ref,
                w00, b00, w10, b10, w20, b20,
                w01, b01, w11, b11, w21, b21,
                out_ref):
    s0 = sq_ref[0, 0, :, 0:_SEL0]
    s1 = sq_ref[0, 1, :, 0:_SEL1]
    desc = jnp.concatenate(
        [1.0 / (jnp.sqrt(s0) + _EPS), 1.0 / (jnp.sqrt(s1) + _EPS)], axis=1)

    def mlp(x, w0, b0, w1, b1, w2, b2):
        h = jnp.tanh(jnp.dot(x, w0[...], preferred_element_type=jnp.float32) + b0[...])
        h = jnp.tanh(jnp.dot(h, w1[...], preferred_element_type=jnp.float32) + b1[...])
        return jnp.dot(h, w2[...], preferred_element_type=jnp.float32) + b2[...]

    e0 = mlp(desc, w00, b00, w10, b10, w20, b20)
    e1 = mlp(desc, w01, b01, w11, b11, w21, b21)
    t = tc_ref[0]
    e = jnp.where(t == 0.0, e0, e1)
    out_ref[0, 0, :] = jnp.broadcast_to(jnp.sum(e), (128,))


def kernel(xyz, box_size, W0_t0, b0_t0, W1_t0, b1_t0, W2_t0, b2_t0,
           W0_t1, b0_t1, W1_t1, b1_t1, W2_t1, b2_t1, atomtypes):
    B, N, _ = xyz.shape
    xt = jnp.transpose(xyz, (0, 2, 1))
    tf = atomtypes.astype(jnp.float32).reshape(B, 1, N)
    tcol = atomtypes.astype(jnp.float32).reshape(B, N, 1)
    box2 = box_size.reshape(1, 1)

    masked = pl.pallas_call(
        _dist_kernel,
        grid=(B, N // _RB),
        in_specs=[
            pl.BlockSpec((1, _RB, 3), lambda b, r: (b, r, 0)),
            pl.BlockSpec((1, 3, N), lambda b, r: (b, 0, 0)),
            pl.BlockSpec((1, 1, N), lambda b, r: (b, 0, 0)),
            pl.BlockSpec((1, 1), lambda b, r: (0, 0)),
        ],
        out_specs=pl.BlockSpec((1, 2, _RB, N), lambda b, r: (b, 0, r, 0)),
        out_shape=jax.ShapeDtypeStruct((B, 2, N, N), jnp.float32),
        compiler_params=pltpu.CompilerParams(
            dimension_semantics=("parallel", "parallel"),
        ),
    )(xyz, xt, tf, box2)

    R = B * 2 * N
    sel = _make_sc_topk(R)(masked.reshape(R * _N))
    sq = sel.reshape(B, 2, N, _K)

    wargs = (W0_t0, b0_t0, W1_t0, b1_t0, W2_t0, b2_t0,
             W0_t1, b0_t1, W1_t1, b1_t1, W2_t1, b2_t1)
    out = pl.pallas_call(
        _mlp_kernel,
        grid=(B,),
        in_specs=[
            pl.BlockSpec((1, 2, N, _K), lambda b: (b, 0, 0, 0)),
            pl.BlockSpec((1, N, 1), lambda b: (b, 0, 0)),
        ] + [pl.BlockSpec(w.shape, functools.partial(
                 lambda nd, b: (0,) * nd, len(w.shape)))
             for w in wargs],
        out_specs=pl.BlockSpec((1, 1, 128), lambda b: (b, 0, 0)),
        out_shape=jax.ShapeDtypeStruct((B, 1, 128), jnp.float32),
    )(sq, tcol, *wargs)

    return out[:, 0, 0]


# speedup vs baseline: 1.1560x; 1.1560x over previous
import functools

import jax
import jax.numpy as jnp
from jax import lax
from jax.experimental import pallas as pl
from jax.experimental.pallas import tpu as pltpu
from jax.experimental.pallas import tpu_sc as plsc

_EPS = 1e-16
_SEL0 = 64
_SEL1 = 128
_D = _SEL0 + _SEL1
_N = 1024
_K = 128
_NREG = _K // 16
_BLK = 16
_RB = 256


def _dist_kernel(xi_ref, xt_ref, tf_ref, box_ref, out_ref):
    rb = pl.program_id(1)
    box = box_ref[0, 0]
    inv_box = 1.0 / box
    xi = xi_ref[0]
    acc = jnp.zeros((_RB, _N), jnp.float32)
    for k in range(3):
        t = xi[:, k : k + 1] - xt_ref[0, k : k + 1, :] + _EPS
        t = t - box * jnp.round(t * inv_box)
        acc = acc + t * t
    row_i = jax.lax.broadcasted_iota(jnp.int32, (_RB, _N), 0) + rb * _RB
    col_j = jax.lax.broadcasted_iota(jnp.int32, (_RB, _N), 1)
    diag = row_i == col_j
    is0 = tf_ref[0] == 0.0
    inf = jnp.float32(jnp.inf)
    out_ref[0, 0] = jnp.where(diag | (~is0), inf, acc)
    out_ref[0, 1] = jnp.where(diag | is0, inf, acc)


def _s16(v):
    r = plsc.sort_key_val(v, v)
    return r[0] if isinstance(r, (tuple, list)) else r


def _bitonic_fix(regs):
    regs = list(regs)
    n = len(regs)
    d = n // 2
    while d >= 1:
        for base in range(0, n, 2 * d):
            for off in range(d):
                a = regs[base + off]
                b = regs[base + off + d]
                regs[base + off] = jnp.minimum(a, b)
                regs[base + off + d] = jnp.maximum(a, b)
        d //= 2
    return [_s16(r) for r in regs]


def _merge(a, b):
    m = len(a)
    fb = [jnp.flip(b[m - 1 - i], 0) for i in range(m)]
    lo = [jnp.minimum(a[i], fb[i]) for i in range(m)]
    hi = [jnp.maximum(a[i], fb[i]) for i in range(m)]
    return _bitonic_fix(lo), _bitonic_fix(hi)


def _sort_chunk(vs):
    rs = [_s16(v) for v in vs]
    width = 1
    while width < len(rs):
        nxt = []
        for p in range(0, len(rs), 2 * width):
            lo, hi = _merge(rs[p : p + width], rs[p + width : p + 2 * width])
            nxt.extend(lo + hi)
        rs = nxt
        width *= 2
    return rs


def _merge_keep_lo(buf, c):
    m = len(buf)
    fc = [jnp.flip(c[m - 1 - i], 0) for i in range(m)]
    lo = [jnp.minimum(buf[i], fc[i]) for i in range(m)]
    return _bitonic_fix(lo)


def _make_sc_topk(R):
    mesh = plsc.VectorSubcoreMesh(core_axis_name="c", subcore_axis_name="s")
    rpw = R // 32
    nblk = rpw // _BLK

    @functools.partial(
        pl.kernel, mesh=mesh,
        compiler_params=pltpu.CompilerParams(needs_layout_passes=False),
        out_type=jax.ShapeDtypeStruct((R * _K,), jnp.float32),
        scratch_types=[
            pltpu.VMEM((_BLK * _N,), jnp.float32),
            pltpu.VMEM((_BLK * _K,), jnp.float32),
        ],
    )
    def k(x_hbm, out_hbm, buf_v, out_v):
        wid = lax.axis_index("s") * 2 + lax.axis_index("c")
        base_row = wid * rpw

        def make_blk_body(nreg_out):
            cregs = nreg_out
            nchunk = (_N // 16) // cregs

            def blk_body(bi, _):
                row0 = base_row + bi * _BLK
                pltpu.sync_copy(x_hbm.at[pl.ds(row0 * _N, _BLK * _N)], buf_v)

                def row_body(j, _):
                    def vreg(t):
                        return buf_v[pl.ds(j * _N + t * 16, 16)]

                    buf = _sort_chunk([vreg(t) for t in range(cregs)])
                    for c in range(1, nchunk):
                        ch = _sort_chunk(
                            [vreg(cregs * c + t) for t in range(cregs)])
                        buf = _merge_keep_lo(buf, ch)
                    for r in range(nreg_out):
                        out_v[pl.ds(j * _K + r * 16, 16)] = buf[r]
                    return 0

                lax.fori_loop(0, _BLK, row_body, 0)
                pltpu.sync_copy(out_v, out_hbm.at[pl.ds(row0 * _K, _BLK * _K)])
                return 0

            return blk_body

        is_t0 = ((wid * rpw) // _N) % 2 == 0

        def run64(_):
            lax.fori_loop(0, nblk, make_blk_body(4), 0)
            return 0

        def run128(_):
            lax.fori_loop(0, nblk, make_blk_body(8), 0)
            return 0

        lax.cond(is_t0, run64, run128, 0)

    return k


def _mlp_kernel(sq_ref, tc_ref,
                w00, b00, w10, b10, w20, b20,
                w01, b01, w11, b11, w21, b21,
                out_ref):
    s0 = sq_ref[0, 0, :, 0:_SEL0]
    s1 = sq_ref[0, 1, :, 0:_SEL1]
    desc = jnp.concatenate(
        [1.0 / (jnp.sqrt(s0) + _EPS), 1.0 / (jnp.sqrt(s1) + _EPS)], axis=1)

    def mlp(x, w0, b0, w1, b1, w2, b2):
        h = jnp.tanh(jnp.dot(x, w0[...], preferred_element_type=jnp.float32) + b0[...])
        h = jnp.tanh(jnp.dot(h, w1[...], preferred_element_type=jnp.float32) + b1[...])
        return jnp.dot(h, w2[...], preferred_element_type=jnp.float32) + b2[...]

    e0 = mlp(desc, w00, b00, w10, b10, w20, b20)
    e1 = mlp(desc, w01, b01, w11, b11, w21, b21)
    t = tc_ref[0]
    e = jnp.where(t == 0.0, e0, e1)
    out_ref[0, 0, :] = jnp.broadcast_to(jnp.sum(e), (128,))


def kernel(xyz, box_size, W0_t0, b0_t0, W1_t0, b1_t0, W2_t0, b2_t0,
           W0_t1, b0_t1, W1_t1, b1_t1, W2_t1, b2_t1, atomtypes):
    B, N, _ = xyz.shape
    xt = jnp.transpose(xyz, (0, 2, 1))
    tf = atomtypes.astype(jnp.float32).reshape(B, 1, N)
    tcol = atomtypes.astype(jnp.float32).reshape(B, N, 1)
    box2 = box_size.reshape(1, 1)

    masked = pl.pallas_call(
        _dist_kernel,
        grid=(B, N // _RB),
        in_specs=[
            pl.BlockSpec((1, _RB, 3), lambda b, r: (b, r, 0)),
            pl.BlockSpec((1, 3, N), lambda b, r: (b, 0, 0)),
            pl.BlockSpec((1, 1, N), lambda b, r: (b, 0, 0)),
            pl.BlockSpec((1, 1), lambda b, r: (0, 0)),
        ],
        out_specs=pl.BlockSpec((1, 2, _RB, N), lambda b, r: (b, 0, r, 0)),
        out_shape=jax.ShapeDtypeStruct((B, 2, N, N), jnp.float32),
        compiler_params=pltpu.CompilerParams(
            dimension_semantics=("parallel", "parallel"),
        ),
    )(xyz, xt, tf, box2)

    R = B * 2 * N
    sel = _make_sc_topk(R)(masked.reshape(R * _N))
    sq = sel.reshape(B, 2, N, _K)

    wargs = (W0_t0, b0_t0, W1_t0, b1_t0, W2_t0, b2_t0,
             W0_t1, b0_t1, W1_t1, b1_t1, W2_t1, b2_t1)
    out = pl.pallas_call(
        _mlp_kernel,
        grid=(B,),
        in_specs=[
            pl.BlockSpec((1, 2, N, _K), lambda b: (b, 0, 0, 0)),
            pl.BlockSpec((1, N, 1), lambda b: (b, 0, 0)),
        ] + [pl.BlockSpec(w.shape, functools.partial(
                 lambda nd, b: (0,) * nd, len(w.shape)))
             for w in wargs],
        out_specs=pl.BlockSpec((1, 1, 128), lambda b: (b, 0, 0)),
        out_shape=jax.ShapeDtypeStruct((B, 1, 128), jnp.float32),
    )(sq, tcol, *wargs)

    return out[:, 0, 0]


# speedup vs baseline: 1.3568x; 1.1737x over previous
import functools

import jax
import jax.numpy as jnp
from jax import lax
from jax.experimental import pallas as pl
from jax.experimental.pallas import tpu as pltpu
from jax.experimental.pallas import tpu_sc as plsc

_EPS = 1e-16
_SEL0 = 64
_SEL1 = 128
_D = _SEL0 + _SEL1
_N = 1024
_K = 128
_NREG = _K // 16
_BLK = 16
_RB = 256


def _dist_kernel(xi_ref, xt_ref, tf_ref, box_ref, out_ref):
    rb = pl.program_id(1)
    box = box_ref[0, 0]
    inv_box = 1.0 / box
    xi = xi_ref[0]
    acc = jnp.zeros((_RB, _N), jnp.float32)
    for k in range(3):
        t = xi[:, k : k + 1] - xt_ref[0, k : k + 1, :] + _EPS
        t = t - box * jnp.round(t * inv_box)
        acc = acc + t * t
    row_i = jax.lax.broadcasted_iota(jnp.int32, (_RB, _N), 0) + rb * _RB
    col_j = jax.lax.broadcasted_iota(jnp.int32, (_RB, _N), 1)
    diag = row_i == col_j
    is0 = tf_ref[0] == 0.0
    inf = jnp.float32(jnp.inf)
    out_ref[0, 0] = jnp.where(diag | (~is0), inf, acc)
    out_ref[0, 1] = jnp.where(diag | is0, inf, acc)


def _s16(v):
    r = plsc.sort_key_val(v, v)
    return r[0] if isinstance(r, (tuple, list)) else r


def _bitonic_fix(regs):
    regs = list(regs)
    n = len(regs)
    d = n // 2
    while d >= 1:
        for base in range(0, n, 2 * d):
            for off in range(d):
                a = regs[base + off]
                b = regs[base + off + d]
                regs[base + off] = jnp.minimum(a, b)
                regs[base + off + d] = jnp.maximum(a, b)
        d //= 2
    return [_s16(r) for r in regs]


def _merge(a, b):
    m = len(a)
    fb = [jnp.flip(b[m - 1 - i], 0) for i in range(m)]
    lo = [jnp.minimum(a[i], fb[i]) for i in range(m)]
    hi = [jnp.maximum(a[i], fb[i]) for i in range(m)]
    return _bitonic_fix(lo), _bitonic_fix(hi)


def _sort_chunk(vs):
    rs = [_s16(v) for v in vs]
    width = 1
    while width < len(rs):
        nxt = []
        for p in range(0, len(rs), 2 * width):
            lo, hi = _merge(rs[p : p + width], rs[p + width : p + 2 * width])
            nxt.extend(lo + hi)
        rs = nxt
        width *= 2
    return rs


def _merge_keep_lo(buf, c):
    m = len(buf)
    fc = [jnp.flip(c[m - 1 - i], 0) for i in range(m)]
    lo = [jnp.minimum(buf[i], fc[i]) for i in range(m)]
    return _bitonic_fix(lo)


def _make_sc_topk(R):
    mesh = plsc.VectorSubcoreMesh(core_axis_name="c", subcore_axis_name="s")
    rpw = R // 32
    nblk = rpw // _BLK

    @functools.partial(
        pl.kernel, mesh=mesh,
        compiler_params=pltpu.CompilerParams(needs_layout_passes=False),
        out_type=jax.ShapeDtypeStruct((R * _K,), jnp.float32),
        scratch_types=[
            pltpu.VMEM((_BLK * _N,), jnp.float32),
            pltpu.VMEM((_BLK * _K,), jnp.float32),
        ],
    )
    def k(x_hbm, out_hbm, buf_v, out_v):
        wid = lax.axis_index("s") * 2 + lax.axis_index("c")
        base_row = wid * rpw

        def blk_body(bi, _):
            row0 = base_row + bi * _BLK
            pltpu.sync_copy(x_hbm.at[pl.ds(row0 * _N, _BLK * _N)], buf_v)

            def row_body(j, _):
                def vreg(t):
                    return buf_v[pl.ds(j * _N + t * 16, 16)]

                buf = _sort_chunk([vreg(t) for t in range(8)])
                for c in range(1, 8):
                    ch = _sort_chunk([vreg(8 * c + t) for t in range(8)])
                    buf = _merge_keep_lo(buf, ch)
                for r in range(_NREG):
                    out_v[pl.ds(j * _K + r * 16, 16)] = buf[r]
                return 0

            lax.fori_loop(0, _BLK, row_body, 0)
            pltpu.sync_copy(out_v, out_hbm.at[pl.ds(row0 * _K, _BLK * _K)])
            return 0

        lax.fori_loop(0, nblk, blk_body, 0)

    return k


def _mlp_kernel(sq_ref, tc_ref,
                w00, b00, w10, b10, w20, b20,
                w01, b01, w11, b11, w21, b21,
                out_ref):
    s0 = sq_ref[0, 0, :, 0:_SEL0]
    s1 = sq_ref[0, 1, :, 0:_SEL1]
    desc = jnp.concatenate(
        [1.0 / (jnp.sqrt(s0) + _EPS), 1.0 / (jnp.sqrt(s1) + _EPS)], axis=1)

    def mlp(x, w0, b0, w1, b1, w2, b2):
        h = jnp.tanh(jnp.dot(x, w0[...], preferred_element_type=jnp.float32) + b0[...])
        h = jnp.tanh(jnp.dot(h, w1[...], preferred_element_type=jnp.float32) + b1[...])
        return jnp.dot(h, w2[...], preferred_element_type=jnp.float32) + b2[...]

    e0 = mlp(desc, w00, b00, w10, b10, w20, b20)
    e1 = mlp(desc, w01, b01, w11, b11, w21, b21)
    t = tc_ref[0]
    e = jnp.where(t == 0.0, e0, e1)
    out_ref[0, 0, :] = jnp.broadcast_to(jnp.sum(e), (128,))


def kernel(xyz, box_size, W0_t0, b0_t0, W1_t0, b1_t0, W2_t0, b2_t0,
           W0_t1, b0_t1, W1_t1, b1_t1, W2_t1, b2_t1, atomtypes):
    B, N, _ = xyz.shape
    xt = jnp.transpose(xyz, (0, 2, 1))
    tf = atomtypes.astype(jnp.float32).reshape(B, 1, N)
    tcol = atomtypes.astype(jnp.float32).reshape(B, N, 1)
    box2 = box_size.reshape(1, 1)

    masked = pl.pallas_call(
        _dist_kernel,
        grid=(B, N // _RB),
        in_specs=[
            pl.BlockSpec((1, _RB, 3), lambda b, r: (b, r, 0)),
            pl.BlockSpec((1, 3, N), lambda b, r: (b, 0, 0)),
            pl.BlockSpec((1, 1, N), lambda b, r: (b, 0, 0)),
            pl.BlockSpec((1, 1), lambda b, r: (0, 0)),
        ],
        out_specs=pl.BlockSpec((1, 2, _RB, N), lambda b, r: (b, 0, r, 0)),
        out_shape=jax.ShapeDtypeStruct((B, 2, N, N), jnp.float32),
        compiler_params=pltpu.CompilerParams(
            dimension_semantics=("parallel", "parallel"),
        ),
    )(xyz, xt, tf, box2)

    R = B * 2 * N
    sel = _make_sc_topk(R)(masked.reshape(R * _N))
    sq = sel.reshape(B, 2, N, _K)

    wargs = (W0_t0, b0_t0, W1_t0, b1_t0, W2_t0, b2_t0,
             W0_t1, b0_t1, W1_t1, b1_t1, W2_t1, b2_t1)
    out = pl.pallas_call(
        _mlp_kernel,
        grid=(B,),
        in_specs=[
            pl.BlockSpec((1, 2, N, _K), lambda b: (b, 0, 0, 0)),
            pl.BlockSpec((1, N, 1), lambda b: (b, 0, 0)),
        ] + [pl.BlockSpec(w.shape, functools.partial(
                 lambda nd, b: (0,) * nd, len(w.shape)))
             for w in wargs],
        out_specs=pl.BlockSpec((1, 1, 128), lambda b: (b, 0, 0)),
        out_shape=jax.ShapeDtypeStruct((B, 1, 128), jnp.float32),
    )(sq, tcol, *wargs)

    return out[:, 0, 0]


# speedup vs baseline: 1.5405x; 1.1354x over previous
import functools

import jax
import jax.numpy as jnp
from jax import lax
from jax.experimental import pallas as pl
from jax.experimental.pallas import tpu as pltpu
from jax.experimental.pallas import tpu_sc as plsc

_EPS = 1e-16
_SEL0 = 64
_SEL1 = 128
_D = _SEL0 + _SEL1
_N = 1024
_K = 128
_NREG = _K // 16
_BLK = 16
_RB = 256


def _dist_kernel(xi_ref, xt_ref, tf_ref, box_ref, out_ref):
    rb = pl.program_id(1)
    box = box_ref[0, 0]
    inv_box = 1.0 / box
    xi = xi_ref[0]
    acc = jnp.zeros((_RB, _N), jnp.float32)
    for k in range(3):
        t = xi[:, k : k + 1] - xt_ref[0, k : k + 1, :] + _EPS
        t = t - box * jnp.round(t * inv_box)
        acc = acc + t * t
    row_i = jax.lax.broadcasted_iota(jnp.int32, (_RB, _N), 0) + rb * _RB
    col_j = jax.lax.broadcasted_iota(jnp.int32, (_RB, _N), 1)
    diag = row_i == col_j
    is0 = tf_ref[0] == 0.0
    inf = jnp.float32(jnp.inf)
    out_ref[0, 0] = jnp.where(diag | (~is0), inf, acc)
    out_ref[0, 1] = jnp.where(diag | is0, inf, acc)


def _s16(v):
    r = plsc.sort_key_val(v, v)
    return r[0] if isinstance(r, (tuple, list)) else r


def _bitonic_fix(regs):
    regs = list(regs)
    n = len(regs)
    d = n // 2
    while d >= 1:
        for base in range(0, n, 2 * d):
            for off in range(d):
                a = regs[base + off]
                b = regs[base + off + d]
                regs[base + off] = jnp.minimum(a, b)
                regs[base + off + d] = jnp.maximum(a, b)
        d //= 2
    return [_s16(r) for r in regs]


def _merge(a, b):
    m = len(a)
    fb = [jnp.flip(b[m - 1 - i], 0) for i in range(m)]
    lo = [jnp.minimum(a[i], fb[i]) for i in range(m)]
    hi = [jnp.maximum(a[i], fb[i]) for i in range(m)]
    return _bitonic_fix(lo), _bitonic_fix(hi)


def _sort_chunk(vs):
    rs = [_s16(v) for v in vs]
    width = 1
    while width < len(rs):
        nxt = []
        for p in range(0, len(rs), 2 * width):
            lo, hi = _merge(rs[p : p + width], rs[p + width : p + 2 * width])
            nxt.extend(lo + hi)
        rs = nxt
        width *= 2
    return rs


def _merge_keep_lo(buf, c):
    m = len(buf)
    fc = [jnp.flip(c[m - 1 - i], 0) for i in range(m)]
    lo = [jnp.minimum(buf[i], fc[i]) for i in range(m)]
    return _bitonic_fix(lo)


def _make_sc_topk(R):
    mesh = plsc.VectorSubcoreMesh(core_axis_name="c", subcore_axis_name="s")
    rpw = R // 32
    nblk = rpw // _BLK

    @functools.partial(
        pl.kernel, mesh=mesh,
        compiler_params=pltpu.CompilerParams(needs_layout_passes=False),
        out_type=jax.ShapeDtypeStruct((R, _K), jnp.float32),
        scratch_types=[
            pltpu.VMEM((_BLK, _N), jnp.float32),
            pltpu.VMEM((_BLK, _K), jnp.float32),
        ],
    )
    def k(x_hbm, out_hbm, buf_v, out_v):
        wid = lax.axis_index("s") * 2 + lax.axis_index("c")
        base_row = wid * rpw

        def blk_body(bi, _):
            row0 = base_row + bi * _BLK
            pltpu.sync_copy(x_hbm.at[pl.ds(row0, _BLK)], buf_v)

            def row_body(j, _):
                def vreg(t):
                    return buf_v[j, pl.ds(t * 16, 16)]

                buf = _sort_chunk([vreg(t) for t in range(8)])
                for c in range(1, 8):
                    ch = _sort_chunk([vreg(8 * c + t) for t in range(8)])
                    buf = _merge_keep_lo(buf, ch)
                for r in range(_NREG):
                    out_v[j, pl.ds(r * 16, 16)] = buf[r]
                return 0

            lax.fori_loop(0, _BLK, row_body, 0)
            pltpu.sync_copy(out_v, out_hbm.at[pl.ds(row0, _BLK)])
            return 0

        lax.fori_loop(0, nblk, blk_body, 0)

    return k


def _mlp_kernel(sq_ref, tc_ref,
                w00, b00, w10, b10, w20, b20,
                w01, b01, w11, b11, w21, b21,
                out_ref):
    s0 = sq_ref[0, 0, :, 0:_SEL0]
    s1 = sq_ref[0, 1, :, 0:_SEL1]
    desc = jnp.concatenate(
        [1.0 / (jnp.sqrt(s0) + _EPS), 1.0 / (jnp.sqrt(s1) + _EPS)], axis=1)

    def mlp(x, w0, b0, w1, b1, w2, b2):
        h = jnp.tanh(jnp.dot(x, w0[...], preferred_element_type=jnp.float32) + b0[...])
        h = jnp.tanh(jnp.dot(h, w1[...], preferred_element_type=jnp.float32) + b1[...])
        return jnp.dot(h, w2[...], preferred_element_type=jnp.float32) + b2[...]

    e0 = mlp(desc, w00, b00, w10, b10, w20, b20)
    e1 = mlp(desc, w01, b01, w11, b11, w21, b21)
    t = tc_ref[0]
    e = jnp.where(t == 0.0, e0, e1)
    out_ref[0, 0, :] = jnp.broadcast_to(jnp.sum(e), (128,))


def kernel(xyz, box_size, W0_t0, b0_t0, W1_t0, b1_t0, W2_t0, b2_t0,
           W0_t1, b0_t1, W1_t1, b1_t1, W2_t1, b2_t1, atomtypes):
    B, N, _ = xyz.shape
    xt = jnp.transpose(xyz, (0, 2, 1))
    tf = atomtypes.astype(jnp.float32).reshape(B, 1, N)
    tcol = atomtypes.astype(jnp.float32).reshape(B, N, 1)
    box2 = box_size.reshape(1, 1)

    masked = pl.pallas_call(
        _dist_kernel,
        grid=(B, N // _RB),
        in_specs=[
            pl.BlockSpec((1, _RB, 3), lambda b, r: (b, r, 0)),
            pl.BlockSpec((1, 3, N), lambda b, r: (b, 0, 0)),
            pl.BlockSpec((1, 1, N), lambda b, r: (b, 0, 0)),
            pl.BlockSpec((1, 1), lambda b, r: (0, 0)),
        ],
        out_specs=pl.BlockSpec((1, 2, _RB, N), lambda b, r: (b, 0, r, 0)),
        out_shape=jax.ShapeDtypeStruct((B, 2, N, N), jnp.float32),
        compiler_params=pltpu.CompilerParams(
            dimension_semantics=("parallel", "parallel"),
        ),
    )(xyz, xt, tf, box2)

    R = B * 2 * N
    sel = _make_sc_topk(R)(masked.reshape(R, _N))
    sq = sel.reshape(B, 2, N, _K)

    wargs = (W0_t0, b0_t0, W1_t0, b1_t0, W2_t0, b2_t0,
             W0_t1, b0_t1, W1_t1, b1_t1, W2_t1, b2_t1)
    out = pl.pallas_call(
        _mlp_kernel,
        grid=(B,),
        in_specs=[
            pl.BlockSpec((1, 2, N, _K), lambda b: (b, 0, 0, 0)),
            pl.BlockSpec((1, N, 1), lambda b: (b, 0, 0)),
        ] + [pl.BlockSpec(w.shape, functools.partial(
                 lambda nd, b: (0,) * nd, len(w.shape)))
             for w in wargs],
        out_specs=pl.BlockSpec((1, 1, 128), lambda b: (b, 0, 0)),
        out_shape=jax.ShapeDtypeStruct((B, 1, 128), jnp.float32),
    )(sq, tcol, *wargs)

    return out[:, 0, 0]


# speedup vs baseline: 1.5895x; 1.0318x over previous
import functools

import jax
import jax.numpy as jnp
from jax import lax
from jax.experimental import pallas as pl
from jax.experimental.pallas import tpu as pltpu
from jax.experimental.pallas import tpu_sc as plsc

_EPS = 1e-16
_SEL0 = 64
_SEL1 = 128
_D = _SEL0 + _SEL1
_N = 1024
_K = 128
_NREG = _K // 16
_BLK = 32
_RB = 256


def _dist_kernel(xi_ref, xt_ref, tf_ref, box_ref, out_ref):
    rb = pl.program_id(1)
    box = box_ref[0, 0]
    inv_box = 1.0 / box
    xi = xi_ref[0]
    acc = jnp.zeros((_RB, _N), jnp.float32)
    for k in range(3):
        t = xi[:, k : k + 1] - xt_ref[0, k : k + 1, :] + _EPS
        t = t - box * jnp.round(t * inv_box)
        acc = acc + t * t
    row_i = jax.lax.broadcasted_iota(jnp.int32, (_RB, _N), 0) + rb * _RB
    col_j = jax.lax.broadcasted_iota(jnp.int32, (_RB, _N), 1)
    diag = row_i == col_j
    is0 = tf_ref[0] == 0.0
    inf = jnp.float32(jnp.inf)
    out_ref[0, 0] = jnp.where(diag | (~is0), inf, acc)
    out_ref[0, 1] = jnp.where(diag | is0, inf, acc)


def _s16(v):
    r = plsc.sort_key_val(v, v)
    return r[0] if isinstance(r, (tuple, list)) else r


def _bitonic_fix(regs):
    regs = list(regs)
    n = len(regs)
    d = n // 2
    while d >= 1:
        for base in range(0, n, 2 * d):
            for off in range(d):
                a = regs[base + off]
                b = regs[base + off + d]
                regs[base + off] = jnp.minimum(a, b)
                regs[base + off + d] = jnp.maximum(a, b)
        d //= 2
    return [_s16(r) for r in regs]


def _merge(a, b):
    m = len(a)
    fb = [jnp.flip(b[m - 1 - i], 0) for i in range(m)]
    lo = [jnp.minimum(a[i], fb[i]) for i in range(m)]
    hi = [jnp.maximum(a[i], fb[i]) for i in range(m)]
    return _bitonic_fix(lo), _bitonic_fix(hi)


def _sort_chunk(vs):
    rs = [_s16(v) for v in vs]
    width = 1
    while width < len(rs):
        nxt = []
        for p in range(0, len(rs), 2 * width):
            lo, hi = _merge(rs[p : p + width], rs[p + width : p + 2 * width])
            nxt.extend(lo + hi)
        rs = nxt
        width *= 2
    return rs


def _merge_keep_lo(buf, c):
    m = len(buf)
    fc = [jnp.flip(c[m - 1 - i], 0) for i in range(m)]
    lo = [jnp.minimum(buf[i], fc[i]) for i in range(m)]
    return _bitonic_fix(lo)


def _make_sc_topk(R):
    mesh = plsc.VectorSubcoreMesh(core_axis_name="c", subcore_axis_name="s")
    rpw = R // 32
    nblk = rpw // _BLK

    @functools.partial(
        pl.kernel, mesh=mesh,
        compiler_params=pltpu.CompilerParams(needs_layout_passes=False),
        out_type=jax.ShapeDtypeStruct((R, _K), jnp.float32),
        scratch_types=[
            pltpu.VMEM((_BLK, _N), jnp.float32),
            pltpu.VMEM((_BLK, _K), jnp.float32),
        ],
    )
    def k(x_hbm, out_hbm, buf_v, out_v):
        wid = lax.axis_index("s") * 2 + lax.axis_index("c")
        base_row = wid * rpw

        def blk_body(bi, _):
            row0 = base_row + bi * _BLK
            pltpu.sync_copy(x_hbm.at[pl.ds(row0, _BLK)], buf_v)

            def row_body(j, _):
                def vreg(t):
                    return buf_v[j, pl.ds(t * 16, 16)]

                buf = _sort_chunk([vreg(t) for t in range(8)])
                for c in range(1, 8):
                    ch = _sort_chunk([vreg(8 * c + t) for t in range(8)])
                    buf = _merge_keep_lo(buf, ch)
                for r in range(_NREG):
                    out_v[j, pl.ds(r * 16, 16)] = buf[r]
                return 0

            lax.fori_loop(0, _BLK, row_body, 0)
            pltpu.sync_copy(out_v, out_hbm.at[pl.ds(row0, _BLK)])
            return 0

        lax.fori_loop(0, nblk, blk_body, 0)

    return k


def _mlp_kernel(sq_ref, tc_ref,
                w00, b00, w10, b10, w20, b20,
                w01, b01, w11, b11, w21, b21,
                out_ref):
    s0 = sq_ref[0, 0, :, 0:_SEL0]
    s1 = sq_ref[0, 1, :, 0:_SEL1]
    desc = jnp.concatenate(
        [1.0 / (jnp.sqrt(s0) + _EPS), 1.0 / (jnp.sqrt(s1) + _EPS)], axis=1)

    def mlp(x, w0, b0, w1, b1, w2, b2):
        h = jnp.tanh(jnp.dot(x, w0[...], preferred_element_type=jnp.float32) + b0[...])
        h = jnp.tanh(jnp.dot(h, w1[...], preferred_element_type=jnp.float32) + b1[...])
        return jnp.dot(h, w2[...], preferred_element_type=jnp.float32) + b2[...]

    e0 = mlp(desc, w00, b00, w10, b10, w20, b20)
    e1 = mlp(desc, w01, b01, w11, b11, w21, b21)
    t = tc_ref[0]
    e = jnp.where(t == 0.0, e0, e1)
    out_ref[0, 0, :] = jnp.broadcast_to(jnp.sum(e), (128,))


def kernel(xyz, box_size, W0_t0, b0_t0, W1_t0, b1_t0, W2_t0, b2_t0,
           W0_t1, b0_t1, W1_t1, b1_t1, W2_t1, b2_t1, atomtypes):
    B, N, _ = xyz.shape
    xt = jnp.transpose(xyz, (0, 2, 1))
    tf = atomtypes.astype(jnp.float32).reshape(B, 1, N)
    tcol = atomtypes.astype(jnp.float32).reshape(B, N, 1)
    box2 = box_size.reshape(1, 1)

    masked = pl.pallas_call(
        _dist_kernel,
        grid=(B, N // _RB),
        in_specs=[
            pl.BlockSpec((1, _RB, 3), lambda b, r: (b, r, 0)),
            pl.BlockSpec((1, 3, N), lambda b, r: (b, 0, 0)),
            pl.BlockSpec((1, 1, N), lambda b, r: (b, 0, 0)),
            pl.BlockSpec((1, 1), lambda b, r: (0, 0)),
        ],
        out_specs=pl.BlockSpec((1, 2, _RB, N), lambda b, r: (b, 0, r, 0)),
        out_shape=jax.ShapeDtypeStruct((B, 2, N, N), jnp.float32),
        compiler_params=pltpu.CompilerParams(
            dimension_semantics=("parallel", "parallel"),
        ),
    )(xyz, xt, tf, box2)

    R = B * 2 * N
    sel = _make_sc_topk(R)(masked.reshape(R, _N))
    sq = sel.reshape(B, 2, N, _K)

    wargs = (W0_t0, b0_t0, W1_t0, b1_t0, W2_t0, b2_t0,
             W0_t1, b0_t1, W1_t1, b1_t1, W2_t1, b2_t1)
    out = pl.pallas_call(
        _mlp_kernel,
        grid=(B,),
        in_specs=[
            pl.BlockSpec((1, 2, N, _K), lambda b: (b, 0, 0, 0)),
            pl.BlockSpec((1, N, 1), lambda b: (b, 0, 0)),
        ] + [pl.BlockSpec(w.shape, functools.partial(
                 lambda nd, b: (0,) * nd, len(w.shape)))
             for w in wargs],
        out_specs=pl.BlockSpec((1, 1, 128), lambda b: (b, 0, 0)),
        out_shape=jax.ShapeDtypeStruct((B, 1, 128), jnp.float32),
    )(sq, tcol, *wargs)

    return out[:, 0, 0]
